# baseline (speedup 1.0000x reference)
import functools

import jax
import jax.numpy as jnp
from jax import lax
from jax.experimental import pallas as pl
from jax.experimental.pallas import tpu as pltpu
from jax.experimental.pallas import tpu_sc as plsc

N = 320000
D = 128
H = 64
S = 10000


ROW_BLK = 10000
N_BLKS = N // (2 * ROW_BLK)


def _sp_contract(x, w1, b1, w2):
    h = jnp.dot(x, w1, preferred_element_type=jnp.float32)
    h = h + b1
    e = jnp.exp(-jnp.abs(h))
    h = jnp.maximum(h, 0.0) + (jnp.log(1.0 + e) - jnp.log(2.0))
    return jnp.sum(h * w2, axis=1, keepdims=True)


def _mlp_body(x0_ref, x1_ref, w1_ref, b1_ref, w2_ref, b2_ref, out_ref):
    w1 = w1_ref[...]
    b1 = b1_ref[...]
    w2 = w2_ref[...]
    b2 = b2_ref[...]
    out_ref[pl.ds(0, ROW_BLK), :] = _sp_contract(x0_ref[...], w1, b1, w2) + b2
    out_ref[pl.ds(ROW_BLK, ROW_BLK), :] = _sp_contract(x1_ref[...], w1, b1, w2) + b2


def _atom_mlp(representation, W1, b1, W2, b2):
    return pl.pallas_call(
        _mlp_body,
        grid=(N_BLKS,),
        in_specs=[
            pl.BlockSpec((ROW_BLK, D), lambda i: (2 * i, 0)),
            pl.BlockSpec((ROW_BLK, D), lambda i: (2 * i + 1, 0)),
            pl.BlockSpec((D, H), lambda i: (0, 0)),
            pl.BlockSpec((1, H), lambda i: (0, 0)),
            pl.BlockSpec((1, H), lambda i: (0, 0)),
            pl.BlockSpec((1, 1), lambda i: (0, 0)),
        ],
        out_specs=pl.BlockSpec((2 * ROW_BLK, 1), lambda i: (i, 0)),
        out_shape=jax.ShapeDtypeStruct((N, 1), jnp.float32),
    )(representation, representation, W1, b1.reshape(1, H),
      W2.reshape(1, H), b2.reshape(1, 1))


L = 16
NSUB = 16
CHUNK = N // NSUB
NVEC = CHUNK // L
SPAD = 10240
OWN = SPAD // NSUB


def _segsum_body(yi_hbm, ids_hbm, out_hbm, vals_v, ids_v, yloc_v, acc_v,
                 tmp_v, shared, sem_a, sem_b):
    sid = lax.axis_index("s")
    base = sid * CHUNK

    cp_a = pltpu.async_copy(
        yi_hbm.at[pl.ds(base, CHUNK)], vals_v.at[pl.ds(0, CHUNK)], sem_a)
    cp_b = pltpu.async_copy(
        ids_hbm.at[pl.ds(base, CHUNK)], ids_v.at[pl.ds(0, CHUNK)], sem_b)

    zeros = jnp.zeros((L,), jnp.float32)

    @plsc.parallel_loop(0, SPAD // L, unroll=8)
    def _(i):
        yloc_v[pl.ds(i * L, L)] = zeros

    cp_a.wait()
    cp_b.wait()
    ids_v[pl.ds(CHUNK, L)] = jnp.zeros((L,), jnp.int32)

    lane = lax.iota(jnp.int32, L)
    last_lane = lane == (L - 1)

    @plsc.parallel_loop(0, NVEC, unroll=8)
    def _(k):
        off = k * L
        x = vals_v[pl.ds(off, L)]
        a = ids_v[pl.ds(off, L)]
        an = ids_v[pl.ds(off + 1, L)]
        c = plsc.cumsum(x)
        isb = a != an
        m1 = isb | last_lane
        m2 = isb & jnp.logical_not(last_lane)
        plsc.addupdate_scatter(yloc_v, [a], c, mask=m1)
        plsc.addupdate_scatter(yloc_v, [an], -c, mask=m2)

    pltpu.sync_copy(yloc_v, shared.at[sid])
    plsc.subcore_barrier()

    pltpu.sync_copy(shared.at[:, pl.ds(sid * OWN, OWN)], tmp_v)

    @plsc.parallel_loop(0, OWN // L, unroll=4)
    def _(i):
        s = tmp_v[0, pl.ds(i * L, L)]
        for t in range(1, NSUB):
            s = s + tmp_v[t, pl.ds(i * L, L)]
        acc_v[pl.ds(i * L, L)] = s

    pltpu.sync_copy(acc_v, out_hbm.at[pl.ds(sid * OWN, OWN)])


def _segment_sum(yi_flat, batch):
    mesh = plsc.VectorSubcoreMesh(
        core_axis_name="c", subcore_axis_name="s", num_cores=1)
    seg = pl.kernel(
        _segsum_body,
        out_type=jax.ShapeDtypeStruct((SPAD,), jnp.float32),
        mesh=mesh,
        scratch_types=[
            pltpu.VMEM((CHUNK,), jnp.float32),
            pltpu.VMEM((CHUNK + L,), jnp.int32),
            pltpu.VMEM((SPAD,), jnp.float32),
            pltpu.VMEM((OWN,), jnp.float32),
            pltpu.VMEM((NSUB, OWN), jnp.float32),
            pltpu.VMEM_SHARED((NSUB, SPAD), jnp.float32),
            pltpu.SemaphoreType.DMA,
            pltpu.SemaphoreType.DMA,
        ],
        compiler_params=pltpu.CompilerParams(needs_layout_passes=False),
    )
    return seg(yi_flat, batch)


def kernel(representation, z, batch, W1, b1, W2, b2):
    yi = _atom_mlp(representation, W1, b1, W2, b2)
    y = _segment_sum(yi.reshape(N), batch)
    return y[:S].reshape(S, 1)


# speedup vs baseline: 1.5993x; 1.5993x over previous
import functools

import jax
import jax.numpy as jnp
from jax import lax
from jax.experimental import pallas as pl
from jax.experimental.pallas import tpu as pltpu
from jax.experimental.pallas import tpu_sc as plsc

N = 320000
D = 128
H = 64
S = 10000


ROW_BLK = 16000
N_BLKS = N // (2 * ROW_BLK)


def _sp_contract(x, w1, b1, w2):
    h = jnp.dot(x, w1, preferred_element_type=jnp.float32)
    h = h + b1
    e = jnp.exp(-jnp.abs(h))
    h = jnp.maximum(h, 0.0) + (jnp.log(1.0 + e) - jnp.log(2.0))
    return lax.dot_general(w2, h, (((1,), (1,)), ((), ())),
                           preferred_element_type=jnp.float32)


def _mlp_body(x0_ref, x1_ref, w1_ref, b1_ref, w2_ref, b2_ref, out_ref):
    w1 = w1_ref[...]
    b1 = b1_ref[...]
    w2 = w2_ref[...]
    b2 = b2_ref[...]
    out_ref[:, pl.ds(0, ROW_BLK)] = _sp_contract(x0_ref[...], w1, b1, w2) + b2
    out_ref[:, pl.ds(ROW_BLK, ROW_BLK)] = _sp_contract(x1_ref[...], w1, b1, w2) + b2


def _atom_mlp(representation, W1, b1, W2, b2):
    return pl.pallas_call(
        _mlp_body,
        grid=(N_BLKS,),
        in_specs=[
            pl.BlockSpec((ROW_BLK, D), lambda i: (2 * i, 0)),
            pl.BlockSpec((ROW_BLK, D), lambda i: (2 * i + 1, 0)),
            pl.BlockSpec((D, H), lambda i: (0, 0)),
            pl.BlockSpec((1, H), lambda i: (0, 0)),
            pl.BlockSpec((1, H), lambda i: (0, 0)),
            pl.BlockSpec((1, 1), lambda i: (0, 0)),
        ],
        out_specs=pl.BlockSpec((1, 2 * ROW_BLK), lambda i: (0, i)),
        out_shape=jax.ShapeDtypeStruct((1, N), jnp.float32),
    )(representation, representation, W1, b1.reshape(1, H),
      W2.reshape(1, H), b2.reshape(1, 1))


L = 16
NSUB = 16
CHUNK = N // NSUB
NVEC = CHUNK // L
SPAD = 10240
OWN = SPAD // NSUB


def _segsum_body(yi_hbm, ids_hbm, out_hbm, vals_v, ids_v, yloc_v, acc_v,
                 tmp_v, shared, sem_a, sem_b):
    sid = lax.axis_index("s")
    base = sid * CHUNK

    cp_a = pltpu.async_copy(
        yi_hbm.at[pl.ds(base, CHUNK)], vals_v.at[pl.ds(0, CHUNK)], sem_a)
    cp_b = pltpu.async_copy(
        ids_hbm.at[pl.ds(base, CHUNK)], ids_v.at[pl.ds(0, CHUNK)], sem_b)

    zeros = jnp.zeros((L,), jnp.float32)

    @plsc.parallel_loop(0, SPAD // L, unroll=8)
    def _(i):
        yloc_v[pl.ds(i * L, L)] = zeros

    cp_a.wait()
    cp_b.wait()
    ids_v[pl.ds(CHUNK, L)] = jnp.zeros((L,), jnp.int32)

    lane = lax.iota(jnp.int32, L)
    last_lane = lane == (L - 1)

    @plsc.parallel_loop(0, NVEC, unroll=4)
    def _(k):
        off = k * L
        x = vals_v[pl.ds(off, L)]
        a = ids_v[pl.ds(off, L)]
        an = ids_v[pl.ds(off + 1, L)]
        c = plsc.cumsum(x)
        isb = a != an
        m1 = isb | last_lane
        m2 = isb & jnp.logical_not(last_lane)
        plsc.addupdate_scatter(yloc_v, [a], c, mask=m1)
        plsc.addupdate_scatter(yloc_v, [an], -c, mask=m2)

    pltpu.sync_copy(yloc_v, shared.at[sid])
    plsc.subcore_barrier()

    pltpu.sync_copy(shared.at[:, pl.ds(sid * OWN, OWN)], tmp_v)

    @plsc.parallel_loop(0, OWN // L, unroll=4)
    def _(i):
        s = tmp_v[0, pl.ds(i * L, L)]
        for t in range(1, NSUB):
            s = s + tmp_v[t, pl.ds(i * L, L)]
        acc_v[pl.ds(i * L, L)] = s

    pltpu.sync_copy(acc_v, out_hbm.at[pl.ds(sid * OWN, OWN)])


def _segment_sum(yi_flat, batch):
    mesh = plsc.VectorSubcoreMesh(
        core_axis_name="c", subcore_axis_name="s", num_cores=1)
    seg = pl.kernel(
        _segsum_body,
        out_type=jax.ShapeDtypeStruct((SPAD,), jnp.float32),
        mesh=mesh,
        scratch_types=[
            pltpu.VMEM((CHUNK,), jnp.float32),
            pltpu.VMEM((CHUNK + L,), jnp.int32),
            pltpu.VMEM((SPAD,), jnp.float32),
            pltpu.VMEM((OWN,), jnp.float32),
            pltpu.VMEM((NSUB, OWN), jnp.float32),
            pltpu.VMEM_SHARED((NSUB, SPAD), jnp.float32),
            pltpu.SemaphoreType.DMA,
            pltpu.SemaphoreType.DMA,
        ],
        compiler_params=pltpu.CompilerParams(needs_layout_passes=False),
    )
    return seg(yi_flat, batch)


def kernel(representation, z, batch, W1, b1, W2, b2):
    yi = _atom_mlp(representation, W1, b1, W2, b2)
    y = _segment_sum(yi.reshape(N), batch)
    return y[:S].reshape(S, 1)


# speedup vs baseline: 1.6505x; 1.0320x over previous
import functools

import jax
import jax.numpy as jnp
from jax import lax
from jax.experimental import pallas as pl
from jax.experimental.pallas import tpu as pltpu
from jax.experimental.pallas import tpu_sc as plsc

N = 320000
D = 128
H = 64
S = 10000


ROW_BLK = 16000
N_BLKS = N // (2 * ROW_BLK)


def _sp_contract(x, w1, b1, w2):
    h = jnp.dot(x, w1, preferred_element_type=jnp.float32)
    h = h + b1
    e = jnp.exp(-jnp.abs(h))
    h = jnp.maximum(h, 0.0) + (jnp.log(1.0 + e) - jnp.log(2.0))
    return lax.dot_general(w2, h, (((1,), (1,)), ((), ())),
                           preferred_element_type=jnp.float32)


def _mlp_body(x0_ref, w1_ref, b1_ref, w2_ref, b2_ref, out_ref):
    w1 = w1_ref[...]
    b1 = b1_ref[...]
    w2 = w2_ref[...]
    b2 = b2_ref[...]
    out_ref[...] = _sp_contract(x0_ref[...], w1, b1, w2) + b2


def _atom_mlp(representation, W1, b1, W2, b2):
    return pl.pallas_call(
        _mlp_body,
        grid=(N_BLKS,),
        in_specs=[
            pl.BlockSpec((2 * ROW_BLK, D), lambda i: (i, 0)),
            pl.BlockSpec((D, H), lambda i: (0, 0)),
            pl.BlockSpec((1, H), lambda i: (0, 0)),
            pl.BlockSpec((1, H), lambda i: (0, 0)),
            pl.BlockSpec((1, 1), lambda i: (0, 0)),
        ],
        out_specs=pl.BlockSpec((1, 2 * ROW_BLK), lambda i: (0, i)),
        out_shape=jax.ShapeDtypeStruct((1, N), jnp.float32),
    )(representation, W1, b1.reshape(1, H),
      W2.reshape(1, H), b2.reshape(1, 1))


L = 16
NSUB = 16
CHUNK = N // NSUB
NVEC = CHUNK // L
SPAD = 10240
OWN = SPAD // NSUB


def _segsum_body(yi_hbm, ids_hbm, out_hbm, vals_v, ids_v, yloc_v, acc_v,
                 tmp_v, shared, sem_a, sem_b):
    sid = lax.axis_index("s")
    base = sid * CHUNK

    cp_a = pltpu.async_copy(
        yi_hbm.at[pl.ds(base, CHUNK)], vals_v.at[pl.ds(0, CHUNK)], sem_a)
    cp_b = pltpu.async_copy(
        ids_hbm.at[pl.ds(base, CHUNK)], ids_v.at[pl.ds(0, CHUNK)], sem_b)

    zeros = jnp.zeros((L,), jnp.float32)

    @plsc.parallel_loop(0, SPAD // L, unroll=8)
    def _(i):
        yloc_v[pl.ds(i * L, L)] = zeros

    cp_a.wait()
    cp_b.wait()
    ids_v[pl.ds(CHUNK, L)] = jnp.zeros((L,), jnp.int32)

    lane = lax.iota(jnp.int32, L)
    last_lane = lane == (L - 1)

    @plsc.parallel_loop(0, NVEC, unroll=4)
    def _(k):
        off = k * L
        x = vals_v[pl.ds(off, L)]
        a = ids_v[pl.ds(off, L)]
        an = ids_v[pl.ds(off + 1, L)]
        c = plsc.cumsum(x)
        isb = a != an
        m1 = isb | last_lane
        m2 = isb & jnp.logical_not(last_lane)
        plsc.addupdate_scatter(yloc_v, [a], c, mask=m1)
        plsc.addupdate_scatter(yloc_v, [an], -c, mask=m2)

    pltpu.sync_copy(yloc_v, shared.at[sid])
    plsc.subcore_barrier()

    pltpu.sync_copy(shared.at[:, pl.ds(sid * OWN, OWN)], tmp_v)

    @plsc.parallel_loop(0, OWN // L, unroll=4)
    def _(i):
        s = tmp_v[0, pl.ds(i * L, L)]
        for t in range(1, NSUB):
            s = s + tmp_v[t, pl.ds(i * L, L)]
        acc_v[pl.ds(i * L, L)] = s

    pltpu.sync_copy(acc_v, out_hbm.at[pl.ds(sid * OWN, OWN)])


def _segment_sum(yi_flat, batch):
    mesh = plsc.VectorSubcoreMesh(
        core_axis_name="c", subcore_axis_name="s", num_cores=1)
    seg = pl.kernel(
        _segsum_body,
        out_type=jax.ShapeDtypeStruct((SPAD,), jnp.float32),
        mesh=mesh,
        scratch_types=[
            pltpu.VMEM((CHUNK,), jnp.float32),
            pltpu.VMEM((CHUNK + L,), jnp.int32),
            pltpu.VMEM((SPAD,), jnp.float32),
            pltpu.VMEM((OWN,), jnp.float32),
            pltpu.VMEM((NSUB, OWN), jnp.float32),
            pltpu.VMEM_SHARED((NSUB, SPAD), jnp.float32),
            pltpu.SemaphoreType.DMA,
            pltpu.SemaphoreType.DMA,
        ],
        compiler_params=pltpu.CompilerParams(needs_layout_passes=False),
    )
    return seg(yi_flat, batch)


def kernel(representation, z, batch, W1, b1, W2, b2):
    yi = _atom_mlp(representation, W1, b1, W2, b2)
    y = _segment_sum(yi.reshape(N), batch)
    return y[:S].reshape(S, 1)
